# baseline (speedup 1.0000x reference)
import functools

import jax
import jax.numpy as jnp
from jax import lax
from jax.experimental import pallas as pl
from jax.experimental.pallas import tpu as pltpu
from jax.experimental.pallas import tpu_sc as plsc

_IMAGE_TOKEN_ID = 32000
_IGNORE_IDX = -100
_PATCH = 14
_IMG = 336
_NPW = _IMG // _PATCH
_NP = _NPW * _NPW
_B = 8
_S = 1024
_D = 2048
_L = _S - 1 + _NP
_KP = 3 * _PATCH * _PATCH
_KPAD = 640

_BM = 512


def _mm_body(x_ref, w_ref, b_ref, ts_ref, to_ref, io_ref,
             o_ref, tso_ref, too_ref, ioo_ref):
    o_ref[...] = (
        jnp.dot(x_ref[...], w_ref[...], preferred_element_type=jnp.float32)
        + b_ref[...]
    )

    @pl.when(pl.program_id(0) == 0)
    def _():
        tso_ref[...] = ts_ref[...]
        too_ref[...] = to_ref[...]
        ioo_ref[...] = io_ref[...]


def _vision_encode(images, W_vis, b_vis, tsrc, tout, iout):
    b = images.shape[0]
    x = images.reshape(b, 3, _NPW, _PATCH, _NPW, _PATCH)
    x = jnp.transpose(x, (0, 2, 4, 1, 3, 5)).reshape(b * _NP, _KP)
    w = W_vis.astype(jnp.float32)
    bb = b_vis.astype(jnp.float32).reshape(1, _D)
    m = b * _NP
    full = lambda s: pl.BlockSpec(s, lambda i: tuple(0 for _ in s))
    return pl.pallas_call(
        _mm_body,
        grid=(m // _BM,),
        in_specs=[
            pl.BlockSpec((_BM, _KP), lambda i: (i, 0)),
            pl.BlockSpec((_KP, _D), lambda i: (0, 0)),
            pl.BlockSpec((1, _D), lambda i: (0, 0)),
            full(tsrc.shape),
            full(tout.shape),
            full(iout.shape),
        ],
        out_specs=[
            pl.BlockSpec((_BM, _D), lambda i: (i, 0)),
            full(tsrc.shape),
            full(tout.shape),
            full(iout.shape),
        ],
        out_shape=[
            jax.ShapeDtypeStruct((m, _D), jnp.float32),
            jax.ShapeDtypeStruct(tsrc.shape, jnp.int32),
            jax.ShapeDtypeStruct(tout.shape, jnp.int32),
            jax.ShapeDtypeStruct(iout.shape, jnp.int32),
        ],
    )(x, w, bb, tsrc, tout, iout)


_NW = 32
_CH = 16
_TROWS = _B * (_S - 1)
_TPAD = 8192
_TPW = _TPAD // _NW
_TNC = _TPW // _CH
_IROWS = _B * _NP
_IPW = _IROWS // _NW
_INC = _IPW // _CH
_NB = 3


def _asm_body(o2_ref, if_ref, fe_ref, im_ref):
    fe_ref[0] = o2_ref[...]
    im_ref[0] = if_ref[...]


def _assemble(out2d, imgf_flat):
    return pl.pallas_call(
        _asm_body,
        grid=(_B, 2),
        in_specs=[
            pl.BlockSpec((800, _D), lambda i, j: (i * 2 + j, 0)),
            pl.BlockSpec((_NP // 2, _D), lambda i, j: (i * 2 + j, 0)),
        ],
        out_specs=[
            pl.BlockSpec((1, 800, _D), lambda i, j: (i, j, 0)),
            pl.BlockSpec((1, _NP // 2, _D), lambda i, j: (i, j, 0)),
        ],
        out_shape=[
            jax.ShapeDtypeStruct((_B, _L, _D), jnp.float32),
            jax.ShapeDtypeStruct((_B, _NP, _D), jnp.float32),
        ],
    )(out2d, imgf_flat)


def _make_splice():
    mesh = plsc.VectorSubcoreMesh(core_axis_name="c", subcore_axis_name="s")

    @functools.partial(
        pl.kernel,
        mesh=mesh,
        compiler_params=pltpu.CompilerParams(use_tc_tiling_on_sc=True),
        out_type=jax.ShapeDtypeStruct((_B * 1600, _D), jnp.float32),
        scratch_types=[
            pltpu.VMEM((_TNC, _CH), jnp.int32),
            pltpu.VMEM((_TNC, _CH), jnp.int32),
            pltpu.VMEM((_INC, _CH), jnp.int32),
        ]
        + [pltpu.VMEM((_CH, _D), jnp.float32) for _ in range(_NB)]
        + [pltpu.SemaphoreType.DMA for _ in range(2 * _NB)],
    )
    def splice(tsrc_hbm, tout_hbm, iout_hbm, embed_hbm, imgf_hbm, out_hbm,
               tsrc_v, tout_v, iout_v, *bufs_and_sems):
        bufs = bufs_and_sems[:_NB]
        g_sems = bufs_and_sems[_NB:2 * _NB]
        s_sems = bufs_and_sems[2 * _NB:]
        wid = lax.axis_index("s") * 2 + lax.axis_index("c")
        pltpu.sync_copy(tsrc_hbm.at[wid], tsrc_v)
        pltpu.sync_copy(tout_hbm.at[wid], tout_v)
        pltpu.sync_copy(iout_hbm.at[wid], iout_v)

        njobs = _TNC + _INC

        def start_gather(c):
            b = c % _NB
            if c < _TNC:
                return pltpu.async_copy(embed_hbm.at[tsrc_v.at[c]],
                                        bufs[b], g_sems[b])
            off = wid * _IPW + (c - _TNC) * _CH
            return pltpu.async_copy(imgf_hbm.at[pl.ds(off, _CH)],
                                    bufs[b], g_sems[b])

        def start_scatter(c):
            b = c % _NB
            idx = tout_v.at[c] if c < _TNC else iout_v.at[c - _TNC]
            return pltpu.async_copy(bufs[b], out_hbm.at[idx], s_sems[b])

        gd = {0: start_gather(0)}
        sd = {}
        for c in range(njobs):
            gd.pop(c).wait()
            sd[c] = start_scatter(c)
            if c + 1 < njobs:
                if c + 1 >= _NB:
                    sd.pop(c + 1 - _NB).wait()
                gd[c + 1] = start_gather(c + 1)
        for c in range(njobs - _NB, njobs):
            if c in sd:
                sd.pop(c).wait()

    return splice


def kernel(input_ids, images, position_ids, attention_mask, labels,
           embed_table, W_vis, b_vis):
    ii = input_ids.astype(jnp.int32)
    img_pos = jnp.argmax(ii == _IMAGE_TOKEN_ID, axis=1).astype(jnp.int32)

    k = jnp.arange(_S - 1, dtype=jnp.int32)[None, :]
    before = k < img_pos[:, None]
    tsrc = jnp.where(before, ii[:, :_S - 1], ii[:, 1:]).reshape(-1)
    rowbase = (jnp.arange(_B, dtype=jnp.int32) * 1600)[:, None]
    tout = (rowbase + k + jnp.where(before, 0, _NP)).reshape(-1)
    pad = _TPAD - _TROWS
    tsrc = jnp.concatenate([tsrc, jnp.broadcast_to(tsrc[-1:], (pad,))])
    tout = jnp.concatenate([tout, jnp.broadcast_to(tout[-1:], (pad,))])
    tsrc = tsrc.reshape(_NW, _TNC, _CH)
    tout = tout.reshape(_NW, _TNC, _CH)

    p = jnp.arange(_NP, dtype=jnp.int32)[None, :]
    iout = (rowbase + img_pos[:, None] + p).reshape(_NW, _INC, _CH)

    imgf_flat, tsrc, tout, iout = _vision_encode(
        images, W_vis, b_vis, tsrc, tout, iout)

    out = _make_splice()(tsrc, tout, iout, embed_table.astype(jnp.float32),
                         imgf_flat)
    final_embeds, image_features = _assemble(out, imgf_flat)

    j = jnp.arange(_L, dtype=jnp.int32)[None, :]
    i = img_pos[:, None]
    is_img = (j >= i) & (j < i + _NP)
    ids_a = jnp.pad(input_ids, ((0, 0), (0, _L - _S)))
    ids_b = jnp.pad(input_ids, ((0, 0), (_L - _S, 0)))
    lab_a = jnp.pad(labels, ((0, 0), (0, _L - _S)))
    lab_b = jnp.pad(labels, ((0, 0), (_L - _S, 0)))
    final_ids = jnp.where(
        j < i, ids_a,
        jnp.where(is_img, jnp.asarray(_IMAGE_TOKEN_ID, input_ids.dtype), ids_b))
    final_labels = jnp.where(
        j < i, lab_a,
        jnp.where(is_img, jnp.asarray(_IGNORE_IDX, labels.dtype), lab_b))
    final_mask = jnp.ones((_B, _L), dtype=bool)
    final_pos = jnp.tile(jnp.arange(_L, dtype=position_ids.dtype)[None, :], (_B, 1))
    return (final_ids, final_pos, final_mask, final_embeds, final_labels,
            image_features)


# speedup vs baseline: 1.0362x; 1.0362x over previous
import functools

import jax
import jax.numpy as jnp
from jax import lax
from jax.experimental import pallas as pl
from jax.experimental.pallas import tpu as pltpu
from jax.experimental.pallas import tpu_sc as plsc

_IMAGE_TOKEN_ID = 32000
_IGNORE_IDX = -100
_PATCH = 14
_IMG = 336
_NPW = _IMG // _PATCH
_NP = _NPW * _NPW
_B = 8
_S = 1024
_D = 2048
_L = _S - 1 + _NP
_KP = 3 * _PATCH * _PATCH
_KPAD = 640

_BM = 576


def _mm_body(x_ref, w_ref, b_ref, ts_ref, to_ref, io_ref,
             o_ref, tso_ref, too_ref, ioo_ref):
    o_ref[0] = (
        jnp.dot(x_ref[...], w_ref[...], preferred_element_type=jnp.float32)
        + b_ref[...]
    )

    @pl.when(pl.program_id(0) == 0)
    def _():
        tso_ref[...] = ts_ref[...]
        too_ref[...] = to_ref[...]
        ioo_ref[...] = io_ref[...]


def _vision_encode(images, W_vis, b_vis, tsrc, tout, iout):
    b = images.shape[0]
    x = images.reshape(b, 3, _NPW, _PATCH, _NPW, _PATCH)
    x = jnp.transpose(x, (0, 2, 4, 1, 3, 5)).reshape(b * _NP, _KP)
    w = W_vis.astype(jnp.float32)
    bb = b_vis.astype(jnp.float32).reshape(1, _D)
    m = b * _NP
    full = lambda s: pl.BlockSpec(s, lambda i: tuple(0 for _ in s))
    return pl.pallas_call(
        _mm_body,
        grid=(m // _BM,),
        in_specs=[
            pl.BlockSpec((_BM, _KP), lambda i: (i, 0)),
            pl.BlockSpec((_KP, _D), lambda i: (0, 0)),
            pl.BlockSpec((1, _D), lambda i: (0, 0)),
            full(tsrc.shape),
            full(tout.shape),
            full(iout.shape),
        ],
        out_specs=[
            pl.BlockSpec((1, _BM, _D), lambda i: (i, 0, 0)),
            full(tsrc.shape),
            full(tout.shape),
            full(iout.shape),
        ],
        out_shape=[
            jax.ShapeDtypeStruct((b, _NP, _D), jnp.float32),
            jax.ShapeDtypeStruct(tsrc.shape, jnp.int32),
            jax.ShapeDtypeStruct(tout.shape, jnp.int32),
            jax.ShapeDtypeStruct(iout.shape, jnp.int32),
        ],
    )(x, w, bb, tsrc, tout, iout)


_NW = 32
_CH = 16
_TROWS = _B * (_S - 1)
_TPAD = 8192
_TPW = _TPAD // _NW
_TNC = _TPW // _CH
_IROWS = _B * _NP
_IPW = _IROWS // _NW
_INC = _IPW // _CH
_NB = 3


def _make_splice():
    mesh = plsc.VectorSubcoreMesh(core_axis_name="c", subcore_axis_name="s")

    @functools.partial(
        pl.kernel,
        mesh=mesh,
        compiler_params=pltpu.CompilerParams(use_tc_tiling_on_sc=True),
        out_type=jax.ShapeDtypeStruct((_B * _L, _D), jnp.float32),
        scratch_types=[
            pltpu.VMEM((_TNC, _CH), jnp.int32),
            pltpu.VMEM((_TNC, _CH), jnp.int32),
            pltpu.VMEM((_INC, _CH), jnp.int32),
        ]
        + [pltpu.VMEM((_CH, _D), jnp.float32) for _ in range(_NB)]
        + [pltpu.SemaphoreType.DMA for _ in range(2 * _NB)],
    )
    def splice(tsrc_hbm, tout_hbm, iout_hbm, embed_hbm, imgf_hbm, out_hbm,
               tsrc_v, tout_v, iout_v, *bufs_and_sems):
        bufs = bufs_and_sems[:_NB]
        g_sems = bufs_and_sems[_NB:2 * _NB]
        s_sems = bufs_and_sems[2 * _NB:]
        wid = lax.axis_index("s") * 2 + lax.axis_index("c")
        pltpu.sync_copy(tsrc_hbm.at[wid], tsrc_v)
        pltpu.sync_copy(tout_hbm.at[wid], tout_v)
        pltpu.sync_copy(iout_hbm.at[wid], iout_v)

        njobs = _TNC + _INC

        def start_gather(c):
            b = c % _NB
            if c < _TNC:
                return pltpu.async_copy(embed_hbm.at[tsrc_v.at[c]],
                                        bufs[b], g_sems[b])
            off = wid * _IPW + (c - _TNC) * _CH
            return pltpu.async_copy(imgf_hbm.at[pl.ds(off, _CH)],
                                    bufs[b], g_sems[b])

        def start_scatter(c):
            b = c % _NB
            idx = tout_v.at[c] if c < _TNC else iout_v.at[c - _TNC]
            return pltpu.async_copy(bufs[b], out_hbm.at[idx], s_sems[b])

        gd = {0: start_gather(0)}
        sd = {}
        for c in range(njobs):
            gd.pop(c).wait()
            sd[c] = start_scatter(c)
            if c + 1 < njobs:
                if c + 1 >= _NB:
                    sd.pop(c + 1 - _NB).wait()
                gd[c + 1] = start_gather(c + 1)
        for c in range(njobs - _NB, njobs):
            if c in sd:
                sd.pop(c).wait()

    return splice


def kernel(input_ids, images, position_ids, attention_mask, labels,
           embed_table, W_vis, b_vis):
    ii = input_ids.astype(jnp.int32)
    img_pos = jnp.argmax(ii == _IMAGE_TOKEN_ID, axis=1).astype(jnp.int32)

    k = jnp.arange(_S - 1, dtype=jnp.int32)[None, :]
    before = k < img_pos[:, None]
    tsrc = jnp.where(before, ii[:, :_S - 1], ii[:, 1:]).reshape(-1)
    rowbase = (jnp.arange(_B, dtype=jnp.int32) * _L)[:, None]
    tout = (rowbase + k + jnp.where(before, 0, _NP)).reshape(-1)
    pad = _TPAD - _TROWS
    tsrc = jnp.concatenate([tsrc, jnp.broadcast_to(tsrc[-1:], (pad,))])
    tout = jnp.concatenate([tout, jnp.broadcast_to(tout[-1:], (pad,))])
    tsrc = tsrc.reshape(_NW, _TNC, _CH)
    tout = tout.reshape(_NW, _TNC, _CH)

    p = jnp.arange(_NP, dtype=jnp.int32)[None, :]
    iout = (rowbase + img_pos[:, None] + p).reshape(_NW, _INC, _CH)

    image_features, tsrc, tout, iout = _vision_encode(
        images, W_vis, b_vis, tsrc, tout, iout)
    imgf_flat = image_features.reshape(_B * _NP, _D)

    out = _make_splice()(tsrc, tout, iout, embed_table.astype(jnp.float32),
                         imgf_flat)
    final_embeds = out.reshape(_B, _L, _D)

    j = jnp.arange(_L, dtype=jnp.int32)[None, :]
    i = img_pos[:, None]
    is_img = (j >= i) & (j < i + _NP)
    ids_a = jnp.pad(input_ids, ((0, 0), (0, _L - _S)))
    ids_b = jnp.pad(input_ids, ((0, 0), (_L - _S, 0)))
    lab_a = jnp.pad(labels, ((0, 0), (0, _L - _S)))
    lab_b = jnp.pad(labels, ((0, 0), (_L - _S, 0)))
    final_ids = jnp.where(
        j < i, ids_a,
        jnp.where(is_img, jnp.asarray(_IMAGE_TOKEN_ID, input_ids.dtype), ids_b))
    final_labels = jnp.where(
        j < i, lab_a,
        jnp.where(is_img, jnp.asarray(_IGNORE_IDX, labels.dtype), lab_b))
    final_mask = jnp.ones((_B, _L), dtype=bool)
    final_pos = jnp.tile(jnp.arange(_L, dtype=position_ids.dtype)[None, :], (_B, 1))
    return (final_ids, final_pos, final_mask, final_embeds, final_labels,
            image_features)
